# Initial kernel scaffold; baseline (speedup 1.0000x reference)
import functools

import jax
import jax.numpy as jnp
from jax import lax
from jax.experimental import pallas as pl
from jax.experimental.pallas import tpu as pltpu
from jax.experimental.pallas import tpu_sc as plsc

N = 10000
E = 320000
D = 128

NC = 2
NS = 16
NW = NC * NS
EPW = E // NW
B = 125
NB = EPW // B
NPT = N // NS
DEGW = 16

_MESH = plsc.VectorSubcoreMesh(
    core_axis_name="c", subcore_axis_name="s", num_cores=NC, num_subcores=NS)


@functools.partial(
    pl.kernel,
    out_type=jax.ShapeDtypeStruct((NC, N, DEGW), jnp.float32),
    mesh=_MESH,
    scratch_types=[
        pltpu.VMEM((NB, B), jnp.int32),
        pltpu.VMEM((B, DEGW), jnp.float32),
        pltpu.VMEM_SHARED((N, DEGW), jnp.float32),
    ],
)
def _deg_kernel(dst_hbm, out_hbm, dst_v, ones_v, acc):
    c = lax.axis_index("c")
    s = lax.axis_index("s")
    wid = s * NC + c
    pltpu.sync_copy(dst_hbm.at[wid], dst_v)

    zero16 = jnp.zeros((DEGW,), jnp.float32)

    def fill(val):
        def body(r, _):
            ones_v[r, :] = val
            return 0
        lax.fori_loop(0, B, body, 0)

    fill(zero16)
    for k in range(NPT // B):
        pltpu.sync_copy(ones_v, acc.at[pl.ds(s * NPT + k * B, B)])
    fill(zero16 + 1.0)
    plsc.subcore_barrier()

    def body(j, _):
        pltpu.sync_copy(ones_v, acc.at[dst_v.at[j]], add=True)
        return 0
    lax.fori_loop(0, NB, body, 0)

    plsc.subcore_barrier()
    pltpu.sync_copy(acc.at[pl.ds(s * NPT, NPT)],
                    out_hbm.at[c, pl.ds(s * NPT, NPT)])


@functools.partial(
    pl.kernel,
    out_type=jax.ShapeDtypeStruct((NC, N, D), jnp.float32),
    mesh=_MESH,
    scratch_types=[
        pltpu.VMEM((NB, B), jnp.int32),
        pltpu.VMEM((NB, B), jnp.int32),
        pltpu.VMEM((B, D), jnp.float32),
        pltpu.VMEM_SHARED((N, D), jnp.float32),
        pltpu.SemaphoreType.DMA,
    ],
)
def _agg_kernel(y_hbm, src_hbm, dst_hbm, out_hbm, src_v, dst_v, rows_v, acc,
                sem):
    c = lax.axis_index("c")
    s = lax.axis_index("s")
    wid = s * NC + c
    pltpu.sync_copy(src_hbm.at[wid], src_v)
    pltpu.sync_copy(dst_hbm.at[wid], dst_v)

    zero16 = jnp.zeros((16,), jnp.float32)

    def zrow(r, _):
        for cc in range(D // 16):
            rows_v[r, pl.ds(cc * 16, 16)] = zero16
        return 0
    lax.fori_loop(0, B, zrow, 0)
    for k in range(NPT // B):
        pltpu.sync_copy(rows_v, acc.at[pl.ds(s * NPT + k * B, B)])
    plsc.subcore_barrier()

    def body(j, _):
        pltpu.async_copy(y_hbm.at[src_v.at[j]], rows_v, sem).wait()
        pltpu.sync_copy(rows_v, acc.at[dst_v.at[j]], add=True)
        return 0
    lax.fori_loop(0, NB, body, 0)

    plsc.subcore_barrier()
    pltpu.sync_copy(acc.at[pl.ds(s * NPT, NPT)],
                    out_hbm.at[c, pl.ds(s * NPT, NPT)])


def _dis_body(p_ref, o_ref):
    deg = p_ref[0:1, :] + p_ref[1:2, :] + 1.0
    o_ref[...] = lax.rsqrt(deg)


def _dis_call(p):
    return pl.pallas_call(
        _dis_body,
        out_shape=jax.ShapeDtypeStruct((1, N * DEGW), jnp.float32),
    )(p)


_RB = 1000
_GRID = N // _RB


def _mm_body(x_ref, w_ref, d_ref, o_ref):
    o_ref[...] = jnp.dot(x_ref[...], w_ref[...],
                         preferred_element_type=jnp.float32) * d_ref[...]


def _mm_call(x, w, dis):
    return pl.pallas_call(
        _mm_body,
        grid=(_GRID,),
        in_specs=[
            pl.BlockSpec((_RB, D), lambda i: (i, 0)),
            pl.BlockSpec((D, D), lambda i: (0, 0)),
            pl.BlockSpec((_RB, 1), lambda i: (i, 0)),
        ],
        out_specs=pl.BlockSpec((_RB, D), lambda i: (i, 0)),
        out_shape=jax.ShapeDtypeStruct((N, D), jnp.float32),
    )(x, w, dis)


def _mid_body(p0_ref, p1_ref, y_ref, d_ref, b_ref, w_ref, o_ref):
    h = (p0_ref[...] + p1_ref[...] + y_ref[...]) * d_ref[...] + b_ref[...]
    h = jnp.maximum(h, 0.0)
    o_ref[...] = jnp.dot(h, w_ref[...],
                         preferred_element_type=jnp.float32) * d_ref[...]


def _mid_call(p0, p1, y, dis, b, w):
    return pl.pallas_call(
        _mid_body,
        grid=(_GRID,),
        in_specs=[
            pl.BlockSpec((_RB, D), lambda i: (i, 0)),
            pl.BlockSpec((_RB, D), lambda i: (i, 0)),
            pl.BlockSpec((_RB, D), lambda i: (i, 0)),
            pl.BlockSpec((_RB, 1), lambda i: (i, 0)),
            pl.BlockSpec((1, D), lambda i: (0, 0)),
            pl.BlockSpec((D, D), lambda i: (0, 0)),
        ],
        out_specs=pl.BlockSpec((_RB, D), lambda i: (i, 0)),
        out_shape=jax.ShapeDtypeStruct((N, D), jnp.float32),
    )(p0, p1, y, dis, b, w)


def _out_body(p0_ref, p1_ref, y_ref, d_ref, b_ref, o_ref):
    o_ref[...] = ((p0_ref[...] + p1_ref[...] + y_ref[...]) * d_ref[...]
                  + b_ref[...])


def _out_call(p0, p1, y, dis, b):
    return pl.pallas_call(
        _out_body,
        grid=(_GRID,),
        in_specs=[
            pl.BlockSpec((_RB, D), lambda i: (i, 0)),
            pl.BlockSpec((_RB, D), lambda i: (i, 0)),
            pl.BlockSpec((_RB, D), lambda i: (i, 0)),
            pl.BlockSpec((_RB, 1), lambda i: (i, 0)),
            pl.BlockSpec((1, D), lambda i: (0, 0)),
        ],
        out_specs=pl.BlockSpec((_RB, D), lambda i: (i, 0)),
        out_shape=jax.ShapeDtypeStruct((N, D), jnp.float32),
    )(p0, p1, y, dis, b)


def kernel(x, edge_index, W1, b1, W2, b2):
    src3 = edge_index[0].reshape(NW, NB, B)
    dst3 = edge_index[1].reshape(NW, NB, B)

    degp = _deg_kernel(dst3)
    dis_row = _dis_call(degp.reshape(NC, N * DEGW))
    dis = dis_row.reshape(N, DEGW)[:, 0:1]

    y1 = _mm_call(x, W1, dis)
    aggp1 = _agg_kernel(y1, src3, dst3)
    y2 = _mid_call(aggp1[0], aggp1[1], y1, dis, b1.reshape(1, D), W2)
    aggp2 = _agg_kernel(y2, src3, dst3)
    return _out_call(aggp2[0], aggp2[1], y2, dis, b2.reshape(1, D))


# speedup vs baseline: 10.6074x; 10.6074x over previous
import functools

import jax
import jax.numpy as jnp
from jax import lax
from jax.experimental import pallas as pl
from jax.experimental.pallas import tpu as pltpu
from jax.experimental.pallas import tpu_sc as plsc

N = 10000
E = 320000
D = 128

NC = 2
NS = 16
NW = NC * NS
EPW = E // NW
B = 125
NBW = EPW // B

NPH = 2
HALF = 5120
NDUMP = 256
ACC_H = HALF + NDUMP
TPO = HALF // NS
TPZ = ACC_H // NS
ZB = 112

DEG_B = 128
DEG_NB = 79
DEG_PAD = DEG_NB * DEG_B - EPW
DEG_SENT = 10367
DEG_ROWS = 768
DEG_TPT = DEG_ROWS // NS
DEGW = 16

_MESH = plsc.VectorSubcoreMesh(
    core_axis_name="c", subcore_axis_name="s", num_cores=NC, num_subcores=NS)


@functools.partial(
    pl.kernel,
    out_type=jax.ShapeDtypeStruct((NC, DEG_ROWS, DEGW), jnp.float32),
    mesh=_MESH,
    scratch_types=[
        pltpu.VMEM((DEG_NB, DEG_B), jnp.int32),
        pltpu.VMEM((DEG_B, DEGW), jnp.float32),
        pltpu.VMEM_SHARED((DEG_ROWS, DEGW), jnp.float32),
    ],
)
def _deg_kernel(row_hbm, oh_hbm, out_hbm, row_v, blk_v, acc):
    c = lax.axis_index("c")
    s = lax.axis_index("s")
    wid = s * NC + c
    pltpu.sync_copy(row_hbm.at[wid], row_v)

    zero16 = jnp.zeros((DEGW,), jnp.float32)

    def zrow(r, _):
        blk_v[r, :] = zero16
        return 0
    lax.fori_loop(0, DEG_B, zrow, 0)
    pltpu.sync_copy(blk_v.at[pl.ds(0, DEG_TPT)],
                    acc.at[pl.ds(s * DEG_TPT, DEG_TPT)])
    plsc.subcore_barrier()

    def body(j, _):
        pltpu.sync_copy(oh_hbm.at[wid, j], blk_v)
        pltpu.sync_copy(blk_v, acc.at[row_v.at[j]], add=True)
        return 0
    lax.fori_loop(0, DEG_NB, body, 0)

    plsc.subcore_barrier()
    pltpu.sync_copy(acc.at[pl.ds(s * DEG_TPT, DEG_TPT)],
                    out_hbm.at[c, pl.ds(s * DEG_TPT, DEG_TPT)])


@functools.partial(
    pl.kernel,
    out_type=jax.ShapeDtypeStruct((NPH, NC, HALF, D), jnp.float32),
    mesh=_MESH,
    scratch_types=[
        pltpu.VMEM((NBW, B), jnp.int32),
        pltpu.VMEM((NPH, NBW, B), jnp.int32),
        pltpu.VMEM((B, D), jnp.float32),
        pltpu.VMEM((ZB, D), jnp.float32),
        pltpu.VMEM_SHARED((ACC_H, D), jnp.float32),
        pltpu.SemaphoreType.DMA,
    ],
)
def _agg_kernel(y_hbm, src_hbm, dmap_hbm, out_hbm, src_v, dmap_v, rows_v,
                zb_v, acc, sem):
    c = lax.axis_index("c")
    s = lax.axis_index("s")
    wid = s * NC + c
    pltpu.sync_copy(src_hbm.at[wid], src_v)
    for p in range(NPH):
        pltpu.sync_copy(dmap_hbm.at[p, wid], dmap_v.at[p])

    zero16 = jnp.zeros((16,), jnp.float32)

    def zrow(r, _):
        for cc in range(D // 16):
            zb_v[r, pl.ds(cc * 16, 16)] = zero16
        return 0
    lax.fori_loop(0, ZB, zrow, 0)

    for p in range(NPH):
        h = (c + p) % NC
        for z in range(TPZ // ZB):
            pltpu.sync_copy(zb_v, acc.at[pl.ds(s * TPZ + z * ZB, ZB)])
        plsc.subcore_barrier()

        def body(j, _):
            pltpu.async_copy(y_hbm.at[src_v.at[j]], rows_v, sem).wait()
            pltpu.sync_copy(rows_v, acc.at[dmap_v.at[p, j]], add=True)
            return 0
        lax.fori_loop(0, NBW, body, 0)

        plsc.subcore_barrier()
        pltpu.sync_copy(acc.at[pl.ds(s * TPO, TPO)],
                        out_hbm.at[p, h, pl.ds(s * TPO, TPO)])
        plsc.subcore_barrier()


def _agg_index_prep(dst):
    dump = HALF + (dst % NDUMP)
    wids = jnp.arange(NW)
    phases = []
    for p in range(NPH):
        maps = []
        for c in range(NC):
            h = (c + p) % NC
            r = dst - h * HALF
            ok = (r >= 0) & (r < HALF)
            maps.append(jnp.where(ok, r, dump))
        m = jnp.stack(maps).reshape(NC, NW, NBW, B)
        phases.append(m[wids % NC, wids])
    return jnp.stack(phases)


def _deg_index_prep(dst):
    d = dst.reshape(NW, EPW)
    d = jnp.pad(d, ((0, 0), (0, DEG_PAD)), constant_values=DEG_SENT)
    d = d.reshape(NW, DEG_NB, DEG_B)
    oh = jax.nn.one_hot(d % DEGW, DEGW, dtype=jnp.float32)
    return d // DEGW, oh


def _dis_body(p_ref, o_ref):
    deg = p_ref[0:1, :] + p_ref[1:2, :] + 1.0
    o_ref[...] = lax.rsqrt(deg)


def _dis_call(p):
    return pl.pallas_call(
        _dis_body,
        out_shape=jax.ShapeDtypeStruct((1, DEG_ROWS * DEGW), jnp.float32),
    )(p)


_RB = 1000
_GRID = N // _RB


def _mm_body(x_ref, w_ref, d_ref, o_ref):
    o_ref[...] = jnp.dot(x_ref[...], w_ref[...],
                         preferred_element_type=jnp.float32) * d_ref[...]


def _mm_call(x, w, dis):
    return pl.pallas_call(
        _mm_body,
        grid=(_GRID,),
        in_specs=[
            pl.BlockSpec((_RB, D), lambda i: (i, 0)),
            pl.BlockSpec((D, D), lambda i: (0, 0)),
            pl.BlockSpec((_RB, 1), lambda i: (i, 0)),
        ],
        out_specs=pl.BlockSpec((_RB, D), lambda i: (i, 0)),
        out_shape=jax.ShapeDtypeStruct((N, D), jnp.float32),
    )(x, w, dis)


def _mid_body(p0_ref, p1_ref, y_ref, d_ref, b_ref, w_ref, o_ref):
    h = (p0_ref[...] + p1_ref[...] + y_ref[...]) * d_ref[...] + b_ref[...]
    h = jnp.maximum(h, 0.0)
    o_ref[...] = jnp.dot(h, w_ref[...],
                         preferred_element_type=jnp.float32) * d_ref[...]


def _mid_call(p0, p1, y, dis, b, w):
    return pl.pallas_call(
        _mid_body,
        grid=(_GRID,),
        in_specs=[
            pl.BlockSpec((_RB, D), lambda i: (i, 0)),
            pl.BlockSpec((_RB, D), lambda i: (i, 0)),
            pl.BlockSpec((_RB, D), lambda i: (i, 0)),
            pl.BlockSpec((_RB, 1), lambda i: (i, 0)),
            pl.BlockSpec((1, D), lambda i: (0, 0)),
            pl.BlockSpec((D, D), lambda i: (0, 0)),
        ],
        out_specs=pl.BlockSpec((_RB, D), lambda i: (i, 0)),
        out_shape=jax.ShapeDtypeStruct((N, D), jnp.float32),
    )(p0, p1, y, dis, b, w)


def _out_body(p0_ref, p1_ref, y_ref, d_ref, b_ref, o_ref):
    o_ref[...] = (p0_ref[...] + p1_ref[...] + y_ref[...]) * d_ref[...] \
        + b_ref[...]


def _out_call(p0, p1, y, dis, b):
    return pl.pallas_call(
        _out_body,
        grid=(_GRID,),
        in_specs=[
            pl.BlockSpec((_RB, D), lambda i: (i, 0)),
            pl.BlockSpec((_RB, D), lambda i: (i, 0)),
            pl.BlockSpec((_RB, D), lambda i: (i, 0)),
            pl.BlockSpec((_RB, 1), lambda i: (i, 0)),
            pl.BlockSpec((1, D), lambda i: (0, 0)),
        ],
        out_specs=pl.BlockSpec((_RB, D), lambda i: (i, 0)),
        out_shape=jax.ShapeDtypeStruct((N, D), jnp.float32),
    )(p0, p1, y, dis, b)


def kernel(x, edge_index, W1, b1, W2, b2):
    src = edge_index[0]
    dst = edge_index[1]
    src2 = src.reshape(NW, NBW, B)
    dmap = _agg_index_prep(dst)
    deg_rows, deg_oh = _deg_index_prep(dst)

    degp = _deg_kernel(deg_rows, deg_oh)
    dis_row = _dis_call(degp.reshape(NC, DEG_ROWS * DEGW))
    dis = dis_row.reshape(DEG_ROWS * DEGW, 1)[:N]

    y1 = _mm_call(x, W1, dis)
    a1 = _agg_kernel(y1, src2, dmap)
    p10 = a1[0].reshape(NC * HALF, D)[:N]
    p11 = a1[1].reshape(NC * HALF, D)[:N]
    y2 = _mid_call(p10, p11, y1, dis, b1.reshape(1, D), W2)
    a2 = _agg_kernel(y2, src2, dmap)
    p20 = a2[0].reshape(NC * HALF, D)[:N]
    p21 = a2[1].reshape(NC * HALF, D)[:N]
    return _out_call(p20, p21, y2, dis, b2.reshape(1, D))


# speedup vs baseline: 11.9681x; 1.1283x over previous
import functools

import jax
import jax.numpy as jnp
from jax import lax
from jax.experimental import pallas as pl
from jax.experimental.pallas import tpu as pltpu
from jax.experimental.pallas import tpu_sc as plsc

N = 10000
E = 320000
D = 128

NC = 2
NS = 16
NW = NC * NS
EPW = E // NW
B = 125
NBW = EPW // B

NPH = 2
HALF = 5120
NDUMP = 256
ACC_H = HALF + NDUMP
TPO = HALF // NS
TPZ = ACC_H // NS
ZB = 112

DEG_B = 128
DEG_NB = 79
DEG_PAD = DEG_NB * DEG_B - EPW
DEG_SENT = 10367
DEG_ROWS = 768
DEG_TPT = DEG_ROWS // NS
DEGW = 16

_MESH = plsc.VectorSubcoreMesh(
    core_axis_name="c", subcore_axis_name="s", num_cores=NC, num_subcores=NS)


@functools.partial(
    pl.kernel,
    out_type=jax.ShapeDtypeStruct((NC, DEG_ROWS, DEGW), jnp.float32),
    mesh=_MESH,
    scratch_types=[
        pltpu.VMEM((DEG_NB, DEG_B), jnp.int32),
        pltpu.VMEM((DEG_B, DEGW), jnp.float32),
        pltpu.VMEM_SHARED((DEG_ROWS, DEGW), jnp.float32),
    ],
)
def _deg_kernel(row_hbm, oh_hbm, out_hbm, row_v, blk_v, acc):
    c = lax.axis_index("c")
    s = lax.axis_index("s")
    wid = s * NC + c
    pltpu.sync_copy(row_hbm.at[wid], row_v)

    zero16 = jnp.zeros((DEGW,), jnp.float32)

    def zrow(r, _):
        blk_v[r, :] = zero16
        return 0
    lax.fori_loop(0, DEG_B, zrow, 0)
    pltpu.sync_copy(blk_v.at[pl.ds(0, DEG_TPT)],
                    acc.at[pl.ds(s * DEG_TPT, DEG_TPT)])
    plsc.subcore_barrier()

    def body(j, _):
        pltpu.sync_copy(oh_hbm.at[wid, j], blk_v)
        pltpu.sync_copy(blk_v, acc.at[row_v.at[j]], add=True)
        return 0
    lax.fori_loop(0, DEG_NB, body, 0)

    plsc.subcore_barrier()
    pltpu.sync_copy(acc.at[pl.ds(s * DEG_TPT, DEG_TPT)],
                    out_hbm.at[c, pl.ds(s * DEG_TPT, DEG_TPT)])


@functools.partial(
    pl.kernel,
    out_type=jax.ShapeDtypeStruct((NPH, NC, HALF, D), jnp.float32),
    mesh=_MESH,
    scratch_types=[
        pltpu.VMEM((NBW + 1, B), jnp.int32),
        pltpu.VMEM((NPH, NBW, B), jnp.int32),
        pltpu.VMEM((B, D), jnp.float32),
        pltpu.VMEM((B, D), jnp.float32),
        pltpu.VMEM((ZB, D), jnp.float32),
        pltpu.VMEM_SHARED((ACC_H, D), jnp.float32),
        pltpu.SemaphoreType.DMA,
        pltpu.SemaphoreType.DMA,
    ],
)
def _agg_kernel(y_hbm, src_hbm, dmap_hbm, out_hbm, src_v, dmap_v, buf0, buf1,
                zb_v, acc, sem0, sem1):
    c = lax.axis_index("c")
    s = lax.axis_index("s")
    wid = s * NC + c
    pltpu.sync_copy(src_hbm.at[wid], src_v)
    for p in range(NPH):
        pltpu.sync_copy(dmap_hbm.at[p, wid], dmap_v.at[p])

    zero16 = jnp.zeros((16,), jnp.float32)

    def zrow(r, _):
        for cc in range(D // 16):
            zb_v[r, pl.ds(cc * 16, 16)] = zero16
        return 0
    lax.fori_loop(0, ZB, zrow, 0)

    for p in range(NPH):
        h = (c + p) % NC
        for z in range(TPZ // ZB):
            pltpu.sync_copy(zb_v, acc.at[pl.ds(s * TPZ + z * ZB, ZB)])
        plsc.subcore_barrier()

        def body(g, _):
            j = 2 * g
            h0 = pltpu.async_copy(y_hbm.at[src_v.at[j]], buf0, sem0)
            h1 = pltpu.async_copy(y_hbm.at[src_v.at[j + 1]], buf1, sem1)
            h0.wait()
            pltpu.sync_copy(buf0, acc.at[dmap_v.at[p, j]], add=True)
            h1.wait()
            pltpu.sync_copy(buf1, acc.at[dmap_v.at[p, j + 1]], add=True)
            return 0
        lax.fori_loop(0, NBW // 2, body, 0)

        plsc.subcore_barrier()
        pltpu.sync_copy(acc.at[pl.ds(s * TPO, TPO)],
                        out_hbm.at[p, h, pl.ds(s * TPO, TPO)])
        plsc.subcore_barrier()


def _agg_index_prep(dst):
    dump = HALF + (dst % NDUMP)
    wids = jnp.arange(NW)
    phases = []
    for p in range(NPH):
        maps = []
        for c in range(NC):
            h = (c + p) % NC
            r = dst - h * HALF
            ok = (r >= 0) & (r < HALF)
            maps.append(jnp.where(ok, r, dump))
        m = jnp.stack(maps).reshape(NC, NW, NBW, B)
        phases.append(m[wids % NC, wids])
    return jnp.stack(phases)


def _deg_index_prep(dst):
    d = dst.reshape(NW, EPW)
    d = jnp.pad(d, ((0, 0), (0, DEG_PAD)), constant_values=DEG_SENT)
    d = d.reshape(NW, DEG_NB, DEG_B)
    oh = jax.nn.one_hot(d % DEGW, DEGW, dtype=jnp.float32)
    return d // DEGW, oh


def _dis_body(p_ref, o_ref):
    deg = p_ref[0:1, :] + p_ref[1:2, :] + 1.0
    o_ref[...] = lax.rsqrt(deg)


def _dis_call(p):
    return pl.pallas_call(
        _dis_body,
        out_shape=jax.ShapeDtypeStruct((1, DEG_ROWS * DEGW), jnp.float32),
    )(p)


_RB = 1000
_GRID = N // _RB


def _mm_body(x_ref, w_ref, d_ref, o_ref):
    o_ref[...] = jnp.dot(x_ref[...], w_ref[...],
                         preferred_element_type=jnp.float32) * d_ref[...]


def _mm_call(x, w, dis):
    return pl.pallas_call(
        _mm_body,
        grid=(_GRID,),
        in_specs=[
            pl.BlockSpec((_RB, D), lambda i: (i, 0)),
            pl.BlockSpec((D, D), lambda i: (0, 0)),
            pl.BlockSpec((_RB, 1), lambda i: (i, 0)),
        ],
        out_specs=pl.BlockSpec((_RB, D), lambda i: (i, 0)),
        out_shape=jax.ShapeDtypeStruct((N, D), jnp.float32),
    )(x, w, dis)


def _mid_body(p0_ref, p1_ref, y_ref, d_ref, b_ref, w_ref, o_ref):
    h = (p0_ref[...] + p1_ref[...] + y_ref[...]) * d_ref[...] + b_ref[...]
    h = jnp.maximum(h, 0.0)
    o_ref[...] = jnp.dot(h, w_ref[...],
                         preferred_element_type=jnp.float32) * d_ref[...]


def _mid_call(p0, p1, y, dis, b, w):
    return pl.pallas_call(
        _mid_body,
        grid=(_GRID,),
        in_specs=[
            pl.BlockSpec((_RB, D), lambda i: (i, 0)),
            pl.BlockSpec((_RB, D), lambda i: (i, 0)),
            pl.BlockSpec((_RB, D), lambda i: (i, 0)),
            pl.BlockSpec((_RB, 1), lambda i: (i, 0)),
            pl.BlockSpec((1, D), lambda i: (0, 0)),
            pl.BlockSpec((D, D), lambda i: (0, 0)),
        ],
        out_specs=pl.BlockSpec((_RB, D), lambda i: (i, 0)),
        out_shape=jax.ShapeDtypeStruct((N, D), jnp.float32),
    )(p0, p1, y, dis, b, w)


def _out_body(p0_ref, p1_ref, y_ref, d_ref, b_ref, o_ref):
    o_ref[...] = (p0_ref[...] + p1_ref[...] + y_ref[...]) * d_ref[...] \
        + b_ref[...]


def _out_call(p0, p1, y, dis, b):
    return pl.pallas_call(
        _out_body,
        grid=(_GRID,),
        in_specs=[
            pl.BlockSpec((_RB, D), lambda i: (i, 0)),
            pl.BlockSpec((_RB, D), lambda i: (i, 0)),
            pl.BlockSpec((_RB, D), lambda i: (i, 0)),
            pl.BlockSpec((_RB, 1), lambda i: (i, 0)),
            pl.BlockSpec((1, D), lambda i: (0, 0)),
        ],
        out_specs=pl.BlockSpec((_RB, D), lambda i: (i, 0)),
        out_shape=jax.ShapeDtypeStruct((N, D), jnp.float32),
    )(p0, p1, y, dis, b)


def kernel(x, edge_index, W1, b1, W2, b2):
    src = edge_index[0]
    dst = edge_index[1]
    src2 = src.reshape(NW, NBW, B)
    src2 = jnp.concatenate([src2, src2[:, :1]], axis=1)
    dmap = _agg_index_prep(dst)
    deg_rows, deg_oh = _deg_index_prep(dst)

    degp = _deg_kernel(deg_rows, deg_oh)
    dis_row = _dis_call(degp.reshape(NC, DEG_ROWS * DEGW))
    dis = dis_row.reshape(DEG_ROWS * DEGW, 1)[:N]

    y1 = _mm_call(x, W1, dis)
    a1 = _agg_kernel(y1, src2, dmap)
    p10 = a1[0].reshape(NC * HALF, D)[:N]
    p11 = a1[1].reshape(NC * HALF, D)[:N]
    y2 = _mid_call(p10, p11, y1, dis, b1.reshape(1, D), W2)
    a2 = _agg_kernel(y2, src2, dmap)
    p20 = a2[0].reshape(NC * HALF, D)[:N]
    p21 = a2[1].reshape(NC * HALF, D)[:N]
    return _out_call(p20, p21, y2, dis, b2.reshape(1, D))
